# baseline (speedup 1.0000x reference)
import functools
import math

import numpy as np
import jax
import jax.numpy as jnp
from jax import lax
from jax.experimental import pallas as pl
from jax.experimental.pallas import tpu as pltpu
from jax.experimental.pallas import tpu_sc as plsc

D_MODEL = 128
MAX_SEQ = 8192
NC, NS = 2, 16
NW = NC * NS
LANES = 16
R = 64
SCALE = math.sqrt(float(D_MODEL))


def _make_pe_np(max_seq, d_model):
    position = np.arange(max_seq, dtype=np.float32)[:, None]
    div_term = np.exp(
        np.arange(0, d_model, 2, dtype=np.float32) * (-math.log(10000.0) / d_model))
    pe = np.zeros((max_seq, d_model), dtype=np.float32)
    pe[:, 0::2] = np.sin(position * div_term)
    pe[:, 1::2] = np.cos(position * div_term)
    return pe


def _pack_pe_np(pe):
    import ml_dtypes
    s, d = pe.shape
    b = pe.astype(ml_dtypes.bfloat16).view(np.uint16).astype(np.uint32)
    g = b.reshape(s, d // 32, 2, 16)
    words = g[:, :, 0, :] | (g[:, :, 1, :] << 16)
    return words.reshape(s, d // 2).astype(np.int32)


@functools.cache
def _build(batch, seq_len, d):
    assert seq_len % NW == 0
    spw = seq_len // NW
    assert spw % R == 0
    nch = spw // R
    mesh = plsc.VectorSubcoreMesh(core_axis_name="c", subcore_axis_name="s")

    @functools.partial(
        pl.kernel,
        mesh=mesh,
        out_type=jax.ShapeDtypeStruct((batch, seq_len, d), jnp.float32),
        scratch_types=[
            pltpu.VMEM((batch, spw), jnp.int32),
            pltpu.VMEM((2, batch, R, d), jnp.float32),
            pltpu.VMEM((spw, d // 2), jnp.int32),
            pltpu.SemaphoreType.DMA,
            pltpu.SemaphoreType.DMA,
            pltpu.SemaphoreType.DMA,
            pltpu.SemaphoreType.DMA,
            pltpu.SemaphoreType.DMA,
        ],
    )
    def emb_kernel(table_hbm, x_hbm, pe_hbm, out_hbm,
                   idx_v, rows_v, pe_v, g0, g1, o0, o1, psem):
        gsem = (g0, g1)
        osem = (o0, o1)
        wid = lax.axis_index("s") * NC + lax.axis_index("c")
        s0 = wid * spw

        pdma = pltpu.async_copy(pe_hbm.at[pl.ds(s0, spw)], pe_v, psem)
        for b in range(batch):
            pltpu.sync_copy(x_hbm.at[b, pl.ds(s0, spw)], idx_v.at[b])

        def gather(h, buf):
            return [pltpu.async_copy(
                table_hbm.at[idx_v.at[b, pl.ds(h * R, R)]], rows_v.at[buf, b],
                gsem[buf]) for b in range(batch)]

        gd = [None] * nch
        od = [None] * nch
        gd[0] = gather(0, 0)
        pdma.wait()
        for h in range(nch):
            bb = h & 1
            nb = bb ^ 1
            if h + 1 < nch:
                if h >= 1:
                    for c in od[h - 1]:
                        c.wait()
                gd[h + 1] = gather(h + 1, nb)
            for c in gd[h]:
                c.wait()

            @plsc.parallel_loop(0, R, unroll=2)
            def comp(i, _bb=bb, _h=h):
                for j in range(d // (2 * LANES)):
                    pw = pe_v[_h * R + i, pl.ds(j * LANES, LANES)]
                    pa = lax.bitcast_convert_type(pw << 16, jnp.float32)
                    pb = lax.bitcast_convert_type(pw & jnp.int32(-65536),
                                                  jnp.float32)
                    sa = pl.ds(j * 2 * LANES, LANES)
                    sb = pl.ds(j * 2 * LANES + LANES, LANES)
                    for b in range(batch):
                        rows_v[_bb, b, i, sa] = rows_v[_bb, b, i, sa] * SCALE + pa
                        rows_v[_bb, b, i, sb] = rows_v[_bb, b, i, sb] * SCALE + pb

            od[h] = [pltpu.async_copy(
                rows_v.at[bb, b], out_hbm.at[b, pl.ds(s0 + h * R, R)], osem[bb])
                for b in range(batch)]
        for h in (nch - 2, nch - 1):
            if 0 <= h:
                for c in od[h]:
                    c.wait()

    return emb_kernel


def kernel(x, table):
    batch, seq_len = x.shape
    d = table.shape[1]
    pe = jnp.asarray(_pack_pe_np(_make_pe_np(MAX_SEQ, d)[:seq_len]))
    return _build(batch, seq_len, d)(table, x.astype(jnp.int32), pe)


# speedup vs baseline: 1.0238x; 1.0238x over previous
import functools
import math

import numpy as np
import jax
import jax.numpy as jnp
from jax import lax
from jax.experimental import pallas as pl
from jax.experimental.pallas import tpu as pltpu
from jax.experimental.pallas import tpu_sc as plsc

D_MODEL = 128
MAX_SEQ = 8192
NC, NS = 2, 16
NW = NC * NS
LANES = 16
R = 128
SCALE = math.sqrt(float(D_MODEL))


def _make_pe_np(max_seq, d_model):
    position = np.arange(max_seq, dtype=np.float32)[:, None]
    div_term = np.exp(
        np.arange(0, d_model, 2, dtype=np.float32) * (-math.log(10000.0) / d_model))
    pe = np.zeros((max_seq, d_model), dtype=np.float32)
    pe[:, 0::2] = np.sin(position * div_term)
    pe[:, 1::2] = np.cos(position * div_term)
    return pe


@functools.cache
def _build(batch, seq_len, d):
    assert seq_len % NW == 0
    spw = seq_len // NW
    assert spw % R == 0
    hpw = spw // R
    nch = batch * hpw
    mesh = plsc.VectorSubcoreMesh(core_axis_name="c", subcore_axis_name="s")

    @functools.partial(
        pl.kernel,
        mesh=mesh,
        out_type=jax.ShapeDtypeStruct((batch, seq_len, d), jnp.float32),
        scratch_types=[
            pltpu.VMEM((batch, spw), jnp.int32),
            pltpu.VMEM((2, R, d), jnp.float32),
            pltpu.VMEM((spw, d), jnp.float32),
            pltpu.SemaphoreType.DMA,
            pltpu.SemaphoreType.DMA,
            pltpu.SemaphoreType.DMA,
            pltpu.SemaphoreType.DMA,
            pltpu.SemaphoreType.DMA,
        ],
    )
    def emb_kernel(table_hbm, x_hbm, pe_hbm, out_hbm,
                   idx_v, rows_v, pe_v, g0, g1, o0, o1, psem):
        gsem = (g0, g1)
        osem = (o0, o1)
        wid = lax.axis_index("s") * NC + lax.axis_index("c")
        s0 = wid * spw

        pdma = pltpu.async_copy(pe_hbm.at[pl.ds(s0, spw)], pe_v, psem)
        for b in range(batch):
            pltpu.sync_copy(x_hbm.at[b, pl.ds(s0, spw)], idx_v.at[b])

        chunks = [(b, h) for b in range(batch) for h in range(hpw)]

        def gather(c, buf):
            b, h = chunks[c]
            return pltpu.async_copy(
                table_hbm.at[idx_v.at[b, pl.ds(h * R, R)]], rows_v.at[buf],
                gsem[buf])

        gd = [None] * nch
        od = [None] * nch
        gd[0] = gather(0, 0)
        pdma.wait()
        for c in range(nch):
            b, h = chunks[c]
            bb = c & 1
            nb = bb ^ 1
            if c + 1 < nch:
                if c >= 1:
                    od[c - 1].wait()
                gd[c + 1] = gather(c + 1, nb)
            gd[c].wait()

            @plsc.parallel_loop(0, R, unroll=2)
            def comp(i, _bb=bb, _h=h):
                for j in range(d // LANES):
                    sl = pl.ds(j * LANES, LANES)
                    rows_v[_bb, i, sl] = (
                        rows_v[_bb, i, sl] * SCALE + pe_v[_h * R + i, sl])

            od[c] = pltpu.async_copy(
                rows_v.at[bb], out_hbm.at[b, pl.ds(s0 + h * R, R)], osem[bb])
        if nch >= 2:
            od[nch - 2].wait()
        od[nch - 1].wait()

    return emb_kernel


def kernel(x, table):
    batch, seq_len = x.shape
    d = table.shape[1]
    pe = jnp.asarray(_make_pe_np(MAX_SEQ, d)[:seq_len])
    return _build(batch, seq_len, d)(table, x.astype(jnp.int32), pe)


# speedup vs baseline: 1.0478x; 1.0234x over previous
import functools
import math

import numpy as np
import jax
import jax.numpy as jnp
from jax import lax
from jax.experimental import pallas as pl
from jax.experimental.pallas import tpu as pltpu
from jax.experimental.pallas import tpu_sc as plsc

D_MODEL = 128
MAX_SEQ = 8192
NC, NS = 2, 16
NW = NC * NS
LANES = 16
R = 128
SCALE = math.sqrt(float(D_MODEL))


def _make_pe_np(max_seq, d_model):
    position = np.arange(max_seq, dtype=np.float32)[:, None]
    div_term = np.exp(
        np.arange(0, d_model, 2, dtype=np.float32) * (-math.log(10000.0) / d_model))
    pe = np.zeros((max_seq, d_model), dtype=np.float32)
    pe[:, 0::2] = np.sin(position * div_term)
    pe[:, 1::2] = np.cos(position * div_term)
    return pe


@functools.cache
def _build(batch, seq_len, d):
    assert seq_len % NW == 0
    spw = seq_len // NW
    assert spw % R == 0
    hpw = spw // R
    nch = batch * hpw
    mesh = plsc.VectorSubcoreMesh(core_axis_name="c", subcore_axis_name="s")

    @functools.partial(
        pl.kernel,
        mesh=mesh,
        out_type=jax.ShapeDtypeStruct((batch, seq_len, d), jnp.float32),
        scratch_types=[
            pltpu.VMEM((batch, spw), jnp.int32),
            pltpu.VMEM((2, R, d), jnp.float32),
            pltpu.VMEM((spw, d), jnp.float32),
            pltpu.SemaphoreType.DMA,
            pltpu.SemaphoreType.DMA,
            pltpu.SemaphoreType.DMA,
            pltpu.SemaphoreType.DMA,
            pltpu.SemaphoreType.DMA,
        ],
    )
    def emb_kernel(table_hbm, x_hbm, pe_hbm, out_hbm,
                   idx_v, rows_v, pe_v, g0, g1, o0, o1, psem):
        gsem = (g0, g1)
        osem = (o0, o1)
        wid = lax.axis_index("s") * NC + lax.axis_index("c")
        s0 = wid * spw

        pdma = pltpu.async_copy(pe_hbm.at[pl.ds(s0, spw)], pe_v, psem)
        for b in range(batch):
            pltpu.sync_copy(x_hbm.at[b, pl.ds(s0, spw)], idx_v.at[b])

        chunks = [(b, h) for b in range(batch) for h in range(hpw)]

        def gather(c, buf):
            b, h = chunks[c]
            return pltpu.async_copy(
                table_hbm.at[idx_v.at[b, pl.ds(h * R, R)]], rows_v.at[buf],
                gsem[buf])

        gd = [None] * nch
        od = [None] * nch
        gd[0] = gather(0, 0)
        pdma.wait()
        for c in range(nch):
            b, h = chunks[c]
            bb = c & 1
            nb = bb ^ 1
            if c + 1 < nch:
                if c >= 1:
                    od[c - 1].wait()
                gd[c + 1] = gather(c + 1, nb)
            gd[c].wait()

            def comp(i, carry, _bb=bb, _h=h):
                for j in range(d // LANES):
                    sl = pl.ds(j * LANES, LANES)
                    rows_v[_bb, i, sl] = (
                        rows_v[_bb, i, sl] * SCALE + pe_v[_h * R + i, sl])
                return carry

            lax.fori_loop(0, R, comp, 0)
            od[c] = pltpu.async_copy(
                rows_v.at[bb], out_hbm.at[b, pl.ds(s0 + h * R, R)], osem[bb])
        if nch >= 2:
            od[nch - 2].wait()
        od[nch - 1].wait()

    return emb_kernel


def kernel(x, table):
    batch, seq_len = x.shape
    d = table.shape[1]
    pe = jnp.asarray(_make_pe_np(MAX_SEQ, d)[:seq_len])
    return _build(batch, seq_len, d)(table, x.astype(jnp.int32), pe)


# speedup vs baseline: 1.0632x; 1.0147x over previous
import functools
import math

import numpy as np
import jax
import jax.numpy as jnp
from jax import lax
from jax.experimental import pallas as pl
from jax.experimental.pallas import tpu as pltpu
from jax.experimental.pallas import tpu_sc as plsc

D_MODEL = 128
MAX_SEQ = 8192
NC, NS = 2, 16
NW = NC * NS
LANES = 16
R = 128
SCALE = math.sqrt(float(D_MODEL))


def _make_pe_np(max_seq, d_model):
    position = np.arange(max_seq, dtype=np.float32)[:, None]
    div_term = np.exp(
        np.arange(0, d_model, 2, dtype=np.float32) * (-math.log(10000.0) / d_model))
    pe = np.zeros((max_seq, d_model), dtype=np.float32)
    pe[:, 0::2] = np.sin(position * div_term)
    pe[:, 1::2] = np.cos(position * div_term)
    return pe


@functools.cache
def _build(batch, seq_len, d):
    assert seq_len % NW == 0
    spw = seq_len // NW
    assert spw % R == 0
    hpw = spw // R
    nch = batch * hpw
    mesh = plsc.VectorSubcoreMesh(core_axis_name="c", subcore_axis_name="s")

    @functools.partial(
        pl.kernel,
        mesh=mesh,
        out_type=jax.ShapeDtypeStruct((batch, seq_len, d), jnp.float32),
        scratch_types=[
            pltpu.VMEM((batch, spw), jnp.int32),
            pltpu.VMEM((3, R, d), jnp.float32),
            pltpu.VMEM((spw, d), jnp.float32),
            pltpu.SemaphoreType.DMA,
            pltpu.SemaphoreType.DMA,
            pltpu.SemaphoreType.DMA,
            pltpu.SemaphoreType.DMA,
            pltpu.SemaphoreType.DMA,
            pltpu.SemaphoreType.DMA,
            pltpu.SemaphoreType.DMA,
        ],
    )
    def emb_kernel(table_hbm, x_hbm, pe_hbm, out_hbm,
                   idx_v, rows_v, pe_v, g0, g1, g2, o0, o1, o2, psem):
        gsem = (g0, g1, g2)
        osem = (o0, o1, o2)
        wid = lax.axis_index("s") * NC + lax.axis_index("c")
        s0 = wid * spw

        pdma = pltpu.async_copy(pe_hbm.at[pl.ds(s0, spw)], pe_v, psem)
        for b in range(batch):
            pltpu.sync_copy(x_hbm.at[b, pl.ds(s0, spw)], idx_v.at[b])

        chunks = [(b, h) for b in range(batch) for h in range(hpw)]

        def gather(c, buf):
            b, h = chunks[c]
            return pltpu.async_copy(
                table_hbm.at[idx_v.at[b, pl.ds(h * R, R)]], rows_v.at[buf],
                gsem[buf])

        gd = [None] * nch
        od = [None] * nch
        gd[0] = gather(0, 0)
        gd[1] = gather(1, 1)
        pdma.wait()
        for c in range(nch):
            b, h = chunks[c]
            bb = c % 3
            if c + 2 < nch:
                nb = (c + 2) % 3
                if c >= 1:
                    od[c - 1].wait()
                gd[c + 2] = gather(c + 2, nb)
            gd[c].wait()

            def comp(i, carry, _bb=bb, _h=h):
                for j in range(d // LANES):
                    sl = pl.ds(j * LANES, LANES)
                    rows_v[_bb, i, sl] = (
                        rows_v[_bb, i, sl] * SCALE + pe_v[_h * R + i, sl])
                return carry

            lax.fori_loop(0, R, comp, 0)
            od[c] = pltpu.async_copy(
                rows_v.at[bb], out_hbm.at[b, pl.ds(s0 + h * R, R)], osem[bb])
        for c in (nch - 3, nch - 2, nch - 1):
            if c >= 0:
                od[c].wait()

    return emb_kernel


def kernel(x, table):
    batch, seq_len = x.shape
    d = table.shape[1]
    pe = jnp.asarray(_make_pe_np(MAX_SEQ, d)[:seq_len])
    return _build(batch, seq_len, d)(table, x.astype(jnp.int32), pe)
